# baseline (speedup 1.0000x reference)
import functools

import jax
import jax.numpy as jnp
from jax import lax
from jax.experimental import pallas as pl
from jax.experimental.pallas import tpu as pltpu
from jax.experimental.pallas import tpu_sc as plsc

B = 16384
D = 64


_NC, _NS = 2, 16
_NW = _NC * _NS
_BPW = B // _NW
_CHUNK = 128
_NCHUNK = _BPW // _CHUNK


def _gather_body(uid_hbm, iid_hbm, utab_hbm, itab_hbm, uout_hbm, iout_hbm,
                 uidx_v, iidx_v, urows_v, irows_v, usem, isem):
    wid = lax.axis_index("s") * _NC + lax.axis_index("c")
    row0 = wid * _NCHUNK
    base = wid * _BPW
    pltpu.sync_copy(uid_hbm.at[pl.ds(row0, _NCHUNK)], uidx_v)
    pltpu.sync_copy(iid_hbm.at[pl.ds(row0, _NCHUNK)], iidx_v)
    ucopies = []
    icopies = []
    for j in range(_NCHUNK):
        ucopies.append(pltpu.async_copy(
            utab_hbm.at[uidx_v.at[j]],
            urows_v.at[pl.ds(j * _CHUNK, _CHUNK)], usem))
        icopies.append(pltpu.async_copy(
            itab_hbm.at[iidx_v.at[j]],
            irows_v.at[pl.ds(j * _CHUNK, _CHUNK)], isem))
    for c in ucopies + icopies:
        c.wait()
    pltpu.sync_copy(urows_v, uout_hbm.at[pl.ds(base, _BPW)])
    pltpu.sync_copy(irows_v, iout_hbm.at[pl.ds(base, _BPW)])


@functools.cache
def _make_gather():
    return pl.kernel(
        _gather_body,
        out_type=(
            jax.ShapeDtypeStruct((B, D), jnp.float32),
            jax.ShapeDtypeStruct((B, D), jnp.float32),
        ),
        mesh=plsc.VectorSubcoreMesh(core_axis_name="c", subcore_axis_name="s",
                                    num_cores=_NC, num_subcores=_NS),
        scratch_types=[
            pltpu.VMEM((_NCHUNK, _CHUNK), jnp.int32),
            pltpu.VMEM((_NCHUNK, _CHUNK), jnp.int32),
            pltpu.VMEM((_BPW, D), jnp.float32),
            pltpu.VMEM((_BPW, D), jnp.float32),
            pltpu.SemaphoreType.DMA,
            pltpu.SemaphoreType.DMA,
        ],
        compiler_params=pltpu.CompilerParams(use_tc_tiling_on_sc=False),
    )


_BSZ = 1024


def _towers_body(ue_ref, ie_ref, uW1, ub1, uW2, ub2, uPW, uPb,
                 iW1, ib1, iW2, ib2, iPW, iPb, rW, rb,
                 uo_ref, io_ref, r_ref):
    ue = ue_ref[...]
    ie = ie_ref[...]
    f32 = jnp.float32
    u = jnp.maximum(jnp.dot(ue, uW1[...], preferred_element_type=f32) + ub1[...], 0.0)
    u = jnp.maximum(jnp.dot(u, uW2[...], preferred_element_type=f32) + ub2[...], 0.0)
    uo = jnp.dot(u, uPW[...], preferred_element_type=f32) + uPb[...]
    it = jnp.maximum(jnp.dot(ie, iW1[...], preferred_element_type=f32) + ib1[...], 0.0)
    it = jnp.maximum(jnp.dot(it, iW2[...], preferred_element_type=f32) + ib2[...], 0.0)
    io = jnp.dot(it, iPW[...], preferred_element_type=f32) + iPb[...]
    uo_ref[...] = uo
    io_ref[...] = io
    inter = uo * io
    r = jnp.sum(inter * rW[...], axis=1, keepdims=True) + rb[...]
    r_ref[...] = jax.nn.sigmoid(r) * 5.0


def _towers(ue, ie, uW1, ub1, uW2, ub2, uPW, uPb,
            iW1, ib1, iW2, ib2, iPW, iPb, rW, rb):
    full = lambda s: pl.BlockSpec(s, lambda i: (0, 0))
    bspec = pl.BlockSpec((_BSZ, D), lambda i: (i, 0))
    H1, H2 = uW1.shape[1], uW2.shape[1]
    return pl.pallas_call(
        _towers_body,
        grid=(B // _BSZ,),
        in_specs=[
            bspec, bspec,
            full((D, H1)), full((1, H1)), full((H1, H2)), full((1, H2)),
            full((H2, D)), full((1, D)),
            full((D, H1)), full((1, H1)), full((H1, H2)), full((1, H2)),
            full((H2, D)), full((1, D)),
            full((1, D)), full((1, 1)),
        ],
        out_specs=[
            bspec, bspec,
            pl.BlockSpec((_BSZ, 1), lambda i: (i, 0)),
        ],
        out_shape=[
            jax.ShapeDtypeStruct((B, D), jnp.float32),
            jax.ShapeDtypeStruct((B, D), jnp.float32),
            jax.ShapeDtypeStruct((B, 1), jnp.float32),
        ],
    )(ue, ie, uW1, ub1, uW2, ub2, uPW, uPb,
      iW1, ib1, iW2, ib2, iPW, iPb, rW, rb)


def kernel(user_id, item_id, user_table, item_table, uW1, ub1, uW2, ub2, uPW, uPb,
           iW1, ib1, iW2, ib2, iPW, iPb, rW, rb):
    ue, ie = _make_gather()(user_id.reshape(B // _CHUNK, _CHUNK),
                            item_id.reshape(B // _CHUNK, _CHUNK),
                            user_table, item_table)
    return _towers(ue, ie,
                   uW1, ub1.reshape(1, -1), uW2, ub2.reshape(1, -1),
                   uPW, uPb.reshape(1, -1),
                   iW1, ib1.reshape(1, -1), iW2, ib2.reshape(1, -1),
                   iPW, iPb.reshape(1, -1),
                   rW.reshape(1, -1), rb.reshape(1, 1))


# speedup vs baseline: 1.1637x; 1.1637x over previous
import functools

import jax
import jax.numpy as jnp
from jax import lax
from jax.experimental import pallas as pl
from jax.experimental.pallas import tpu as pltpu
from jax.experimental.pallas import tpu_sc as plsc

B = 16384
D = 64
V = 1000000


_NC, _NS = 2, 16
_NW = _NC * _NS
_BPW = B // _NW
_CHUNK = 128
_NCHUNK = _BPW // _CHUNK


def _gather_one(idx_v, tab_hbm, out_hbm, base, sem):
    def group_body(g, carry):
        v = idx_v[pl.ds(g * 16, 16)]
        t = lax.shift_right_logical(v, 3)
        r = lax.bitwise_and(v, 7)
        for l in range(16):
            pltpu.async_copy(tab_hbm.at[t[l], r[l]],
                             out_hbm.at[base + g * 16 + l], sem)
        return carry

    lax.fori_loop(0, _BPW // 16, group_body, 0)
    pltpu.make_async_copy(out_hbm.at[pl.ds(base, _BPW)],
                          out_hbm.at[pl.ds(base, _BPW)], sem).wait()


def _gather_body(uid_hbm, iid_hbm, utab_hbm, itab_hbm, uout_hbm, iout_hbm,
                 uidx_v, iidx_v, sem, sem2):
    wid = lax.axis_index("s") * _NC + lax.axis_index("c")
    base = wid * _BPW
    pltpu.sync_copy(uid_hbm.at[pl.ds(base, _BPW)], uidx_v)
    pltpu.sync_copy(iid_hbm.at[pl.ds(base, _BPW)], iidx_v)
    _gather_one(uidx_v, utab_hbm, uout_hbm, base, sem)
    _gather_one(iidx_v, itab_hbm, iout_hbm, base, sem2)


@functools.cache
def _make_gather():
    return pl.kernel(
        _gather_body,
        out_type=(
            jax.ShapeDtypeStruct((B, D), jnp.float32),
            jax.ShapeDtypeStruct((B, D), jnp.float32),
        ),
        mesh=plsc.VectorSubcoreMesh(core_axis_name="c", subcore_axis_name="s",
                                    num_cores=_NC, num_subcores=_NS),
        scratch_types=[
            pltpu.VMEM((_BPW,), jnp.int32),
            pltpu.VMEM((_BPW,), jnp.int32),
            pltpu.SemaphoreType.DMA,
            pltpu.SemaphoreType.DMA,
        ],
    )


_BSZ = 1024


def _towers_body(ue_ref, ie_ref, uW1, ub1, uW2, ub2, uPW, uPb,
                 iW1, ib1, iW2, ib2, iPW, iPb, rW, rb,
                 uo_ref, io_ref, r_ref):
    ue = ue_ref[...]
    ie = ie_ref[...]
    f32 = jnp.float32
    u = jnp.maximum(jnp.dot(ue, uW1[...], preferred_element_type=f32) + ub1[...], 0.0)
    u = jnp.maximum(jnp.dot(u, uW2[...], preferred_element_type=f32) + ub2[...], 0.0)
    uo = jnp.dot(u, uPW[...], preferred_element_type=f32) + uPb[...]
    it = jnp.maximum(jnp.dot(ie, iW1[...], preferred_element_type=f32) + ib1[...], 0.0)
    it = jnp.maximum(jnp.dot(it, iW2[...], preferred_element_type=f32) + ib2[...], 0.0)
    io = jnp.dot(it, iPW[...], preferred_element_type=f32) + iPb[...]
    uo_ref[...] = uo
    io_ref[...] = io
    inter = uo * io
    r = jnp.sum(inter * rW[...], axis=1, keepdims=True) + rb[...]
    r_ref[...] = jax.nn.sigmoid(r) * 5.0


def _towers(ue, ie, uW1, ub1, uW2, ub2, uPW, uPb,
            iW1, ib1, iW2, ib2, iPW, iPb, rW, rb):
    full = lambda s: pl.BlockSpec(s, lambda i: (0, 0))
    bspec = pl.BlockSpec((_BSZ, D), lambda i: (i, 0))
    H1, H2 = uW1.shape[1], uW2.shape[1]
    return pl.pallas_call(
        _towers_body,
        grid=(B // _BSZ,),
        in_specs=[
            bspec, bspec,
            full((D, H1)), full((1, H1)), full((H1, H2)), full((1, H2)),
            full((H2, D)), full((1, D)),
            full((D, H1)), full((1, H1)), full((H1, H2)), full((1, H2)),
            full((H2, D)), full((1, D)),
            full((1, D)), full((1, 1)),
        ],
        out_specs=[
            bspec, bspec,
            pl.BlockSpec((_BSZ, 1), lambda i: (i, 0)),
        ],
        out_shape=[
            jax.ShapeDtypeStruct((B, D), jnp.float32),
            jax.ShapeDtypeStruct((B, D), jnp.float32),
            jax.ShapeDtypeStruct((B, 1), jnp.float32),
        ],
    )(ue, ie, uW1, ub1, uW2, ub2, uPW, uPb,
      iW1, ib1, iW2, ib2, iPW, iPb, rW, rb)


def kernel(user_id, item_id, user_table, item_table, uW1, ub1, uW2, ub2, uPW, uPb,
           iW1, ib1, iW2, ib2, iPW, iPb, rW, rb):
    ue, ie = _make_gather()(user_id, item_id,
                            user_table.reshape(V // 8, 8, D),
                            item_table.reshape(V // 8, 8, D))
    return _towers(ue, ie,
                   uW1, ub1.reshape(1, -1), uW2, ub2.reshape(1, -1),
                   uPW, uPb.reshape(1, -1),
                   iW1, ib1.reshape(1, -1), iW2, ib2.reshape(1, -1),
                   iPW, iPb.reshape(1, -1),
                   rW.reshape(1, -1), rb.reshape(1, 1))


# speedup vs baseline: 1.5460x; 1.3285x over previous
import functools

import jax
import jax.numpy as jnp
from jax import lax
from jax.experimental import pallas as pl
from jax.experimental.pallas import tpu as pltpu
from jax.experimental.pallas import tpu_sc as plsc

B = 16384
D = 64


_NC, _NS = 2, 16
_NW = _NC * _NS
_BPW = B // _NW


def _gather_one(idx_v, tab_hbm, out_hbm, base, out_v, sem):
    def group_body(g, carry):
        v = idx_v[pl.ds(g * 16, 16)]
        for l in range(16):
            pltpu.async_copy(tab_hbm.at[v[l]],
                             out_v.at[g * 16 + l, pl.ds(0, D)], sem)
        return carry

    lax.fori_loop(0, _BPW // 16, group_body, 0)
    pltpu.make_async_copy(out_hbm.at[pl.ds(base, _BPW // 2)],
                          out_v.at[pl.ds(0, _BPW // 2)], sem).wait()
    pltpu.sync_copy(out_v, out_hbm.at[pl.ds(base, _BPW)])


def _gather_body(uid_hbm, iid_hbm, utab_hbm, itab_hbm, uout_hbm, iout_hbm,
                 uidx_v, iidx_v, out_v, sem, sem2):
    wid = lax.axis_index("s") * _NC + lax.axis_index("c")
    base = wid * _BPW
    pltpu.sync_copy(uid_hbm.at[pl.ds(base, _BPW)], uidx_v)
    pltpu.sync_copy(iid_hbm.at[pl.ds(base, _BPW)], iidx_v)
    _gather_one(uidx_v, utab_hbm, uout_hbm, base, out_v, sem)
    _gather_one(iidx_v, itab_hbm, iout_hbm, base, out_v, sem2)


@functools.cache
def _make_gather():
    return pl.kernel(
        _gather_body,
        out_type=(
            jax.ShapeDtypeStruct((B, 128), jnp.float32),
            jax.ShapeDtypeStruct((B, 128), jnp.float32),
        ),
        mesh=plsc.VectorSubcoreMesh(core_axis_name="c", subcore_axis_name="s",
                                    num_cores=_NC, num_subcores=_NS),
        scratch_types=[
            pltpu.VMEM((_BPW,), jnp.int32),
            pltpu.VMEM((_BPW,), jnp.int32),
            pltpu.VMEM((_BPW, 128), jnp.float32),
            pltpu.SemaphoreType.DMA,
            pltpu.SemaphoreType.DMA,
        ],
    )


_BSZ = 1024


def _towers_body(ue_ref, ie_ref, uW1, ub1, uW2, ub2, uPW, uPb,
                 iW1, ib1, iW2, ib2, iPW, iPb, rW, rb,
                 uo_ref, io_ref, r_ref):
    ue = ue_ref[:, :D]
    ie = ie_ref[:, :D]
    f32 = jnp.float32
    u = jnp.maximum(jnp.dot(ue, uW1[...], preferred_element_type=f32) + ub1[...], 0.0)
    u = jnp.maximum(jnp.dot(u, uW2[...], preferred_element_type=f32) + ub2[...], 0.0)
    uo = jnp.dot(u, uPW[...], preferred_element_type=f32) + uPb[...]
    it = jnp.maximum(jnp.dot(ie, iW1[...], preferred_element_type=f32) + ib1[...], 0.0)
    it = jnp.maximum(jnp.dot(it, iW2[...], preferred_element_type=f32) + ib2[...], 0.0)
    io = jnp.dot(it, iPW[...], preferred_element_type=f32) + iPb[...]
    uo_ref[...] = uo
    io_ref[...] = io
    inter = uo * io
    r = jnp.sum(inter * rW[...], axis=1, keepdims=True) + rb[...]
    r_ref[...] = jax.nn.sigmoid(r) * 5.0


def _towers(ue, ie, uW1, ub1, uW2, ub2, uPW, uPb,
            iW1, ib1, iW2, ib2, iPW, iPb, rW, rb):
    full = lambda s: pl.BlockSpec(s, lambda i: (0, 0))
    bspec = pl.BlockSpec((_BSZ, 128), lambda i: (i, 0))
    ospec = pl.BlockSpec((_BSZ, D), lambda i: (i, 0))
    H1, H2 = uW1.shape[1], uW2.shape[1]
    return pl.pallas_call(
        _towers_body,
        grid=(B // _BSZ,),
        in_specs=[
            bspec, bspec,
            full((D, H1)), full((1, H1)), full((H1, H2)), full((1, H2)),
            full((H2, D)), full((1, D)),
            full((D, H1)), full((1, H1)), full((H1, H2)), full((1, H2)),
            full((H2, D)), full((1, D)),
            full((1, D)), full((1, 1)),
        ],
        out_specs=[
            ospec, ospec,
            pl.BlockSpec((_BSZ, 1), lambda i: (i, 0)),
        ],
        out_shape=[
            jax.ShapeDtypeStruct((B, D), jnp.float32),
            jax.ShapeDtypeStruct((B, D), jnp.float32),
            jax.ShapeDtypeStruct((B, 1), jnp.float32),
        ],
    )(ue, ie, uW1, ub1, uW2, ub2, uPW, uPb,
      iW1, ib1, iW2, ib2, iPW, iPb, rW, rb)


def kernel(user_id, item_id, user_table, item_table, uW1, ub1, uW2, ub2, uPW, uPb,
           iW1, ib1, iW2, ib2, iPW, iPb, rW, rb):
    ue, ie = _make_gather()(user_id, item_id, user_table, item_table)
    return _towers(ue, ie,
                   uW1, ub1.reshape(1, -1), uW2, ub2.reshape(1, -1),
                   uPW, uPb.reshape(1, -1),
                   iW1, ib1.reshape(1, -1), iW2, ib2.reshape(1, -1),
                   iPW, iPb.reshape(1, -1),
                   rW.reshape(1, -1), rb.reshape(1, 1))
